# baseline (speedup 1.0000x reference)
import functools

import jax
import jax.numpy as jnp
import numpy as np
from jax import lax
from jax.experimental import pallas as pl
from jax.experimental.pallas import tpu as pltpu
from jax.experimental.pallas import tpu_sc as plsc

D_MODEL = 256
N_HEADS = 8
N_LEVELS = 4
N_POINTS = 4
N_LAYERS = 6
D_FFN = 1024
SHAPES = [(64, 64), (32, 32), (16, 16), (8, 8)]
LQ = sum(h * w for h, w in SHAPES)
B = 2
DH = D_MODEL // N_HEADS
NPT = N_LEVELS * N_POINTS

NB = 8
T = LQ // NB

N_TILES = 32
TILES_PER_B = N_TILES // B
QPT = LQ // TILES_PER_B
CQ = 2
NCH = QPT // CQ
CHQH = CQ * N_HEADS
ROWS = CHQH * 64
IDX_PER_DMA = 128
NDMA = ROWS // IDX_PER_DMA

_LVL = np.repeat(np.arange(N_LEVELS), N_POINTS)
_W = np.array([s[1] for s in SHAPES], np.float32)[_LVL]
_H = np.array([s[0] for s in SHAPES], np.float32)[_LVL]
_LS = np.array([0, 4096, 5120, 5376], np.int32)[_LVL]


def _lane_const_f(vals):
    return jnp.asarray(vals, jnp.float32)


def _lane_const_i(vals):
    return jnp.asarray(vals, jnp.int32)


def _proj_body(src_ref, pos_ref, rp_ref, wv_ref, bv_ref, wo_ref, bo_ref,
               wa_ref, ba_ref, val_ref, sloc_ref, aw_ref):
    s = src_ref[0]
    q = s + pos_ref[0]
    val_ref[0] = jnp.dot(s, wv_ref[...], preferred_element_type=jnp.float32) + bv_ref[0]
    sloc_ref[0] = rp_ref[0] + jnp.dot(
        q, wo_ref[...], preferred_element_type=jnp.float32) + bo_ref[0]
    logits = jnp.dot(q, wa_ref[...], preferred_element_type=jnp.float32) + ba_ref[0]
    lg = logits.reshape(T, N_HEADS, NPT)
    m = jnp.max(lg, axis=-1, keepdims=True)
    e = jnp.exp(lg - m)
    aw = e / jnp.sum(e, axis=-1, keepdims=True)
    aw_ref[0] = aw.reshape(T, N_HEADS * NPT)


def _proj_call(src, pos, rp_flat, wv, bv, wo_p, bo_p, wa, ba):
    tok = lambda b, i: (b, i, 0)
    fixed = lambda b, i: (0, 0)
    return pl.pallas_call(
        _proj_body,
        grid=(B, NB),
        in_specs=[
            pl.BlockSpec((1, T, D_MODEL), tok),
            pl.BlockSpec((1, T, D_MODEL), tok),
            pl.BlockSpec((1, T, D_MODEL), tok),
            pl.BlockSpec((D_MODEL, D_MODEL), fixed),
            pl.BlockSpec((1, D_MODEL), fixed),
            pl.BlockSpec((D_MODEL, D_MODEL), fixed),
            pl.BlockSpec((1, D_MODEL), fixed),
            pl.BlockSpec((D_MODEL, N_HEADS * NPT), fixed),
            pl.BlockSpec((1, N_HEADS * NPT), fixed),
        ],
        out_specs=[
            pl.BlockSpec((1, T, D_MODEL), tok),
            pl.BlockSpec((1, T, D_MODEL), tok),
            pl.BlockSpec((1, T, N_HEADS * NPT), tok),
        ],
        out_shape=[
            jax.ShapeDtypeStruct((B, LQ, D_MODEL), jnp.float32),
            jax.ShapeDtypeStruct((B, LQ, D_MODEL), jnp.float32),
            jax.ShapeDtypeStruct((B, LQ, N_HEADS * NPT), jnp.float32),
        ],
    )(src, pos, rp_flat, wv, bv, wo_p, bo_p, wa, ba)


def _post_body(attn_ref, src_ref, wo_ref, bo_ref, g1_ref, b1_ref,
               w1_ref, bf1_ref, w2_ref, bf2_ref, g2_ref, b2_ref, out_ref):
    a = (jnp.dot(attn_ref[0], wo_ref[...], preferred_element_type=jnp.float32)
         + bo_ref[0] + src_ref[0])
    mu = jnp.mean(a, axis=-1, keepdims=True)
    var = jnp.mean(jnp.square(a - mu), axis=-1, keepdims=True)
    s2 = (a - mu) * jax.lax.rsqrt(var + 1e-5) * g1_ref[0] + b1_ref[0]
    h = jnp.maximum(
        jnp.dot(s2, w1_ref[...], preferred_element_type=jnp.float32) + bf1_ref[0], 0.0)
    f = (jnp.dot(h, w2_ref[...], preferred_element_type=jnp.float32)
         + bf2_ref[0] + s2)
    mu2 = jnp.mean(f, axis=-1, keepdims=True)
    var2 = jnp.mean(jnp.square(f - mu2), axis=-1, keepdims=True)
    out_ref[0] = (f - mu2) * jax.lax.rsqrt(var2 + 1e-5) * g2_ref[0] + b2_ref[0]


def _post_call(attn, src, wo, bo, g1, b1, w1, bf1, w2, bf2, g2, b2):
    tok = lambda b, i: (b, i, 0)
    fixed = lambda b, i: (0, 0)
    return pl.pallas_call(
        _post_body,
        grid=(B, NB),
        in_specs=[
            pl.BlockSpec((1, T, D_MODEL), tok),
            pl.BlockSpec((1, T, D_MODEL), tok),
            pl.BlockSpec((D_MODEL, D_MODEL), fixed),
            pl.BlockSpec((1, D_MODEL), fixed),
            pl.BlockSpec((1, D_MODEL), fixed),
            pl.BlockSpec((1, D_MODEL), fixed),
            pl.BlockSpec((D_MODEL, D_FFN), fixed),
            pl.BlockSpec((1, D_FFN), fixed),
            pl.BlockSpec((D_FFN, D_MODEL), fixed),
            pl.BlockSpec((1, D_MODEL), fixed),
            pl.BlockSpec((1, D_MODEL), fixed),
            pl.BlockSpec((1, D_MODEL), fixed),
        ],
        out_specs=pl.BlockSpec((1, T, D_MODEL), tok),
        out_shape=jax.ShapeDtypeStruct((B, LQ, D_MODEL), jnp.float32),
    )(attn, src, wo, bo, g1, b1, w1, bf1, w2, bf2, g2, b2)


def _sample_body(val_hbm, sloc_hbm, aw_hbm, out_hbm,
                 idx_v, w_v, rows_v, sloc_v, aw_v, out_v,
                 sem_in0, sem_in1, sem_g0, sem_g1, sem_o0, sem_o1):
    sem_in = (sem_in0, sem_in1)
    sem_g = (sem_g0, sem_g1)
    sem_o = (sem_o0, sem_o1)
    wid = lax.axis_index("s") * 2 + lax.axis_index("c")
    b = wid % 2
    q0 = (wid // 2) * QPT

    lvl = lax.shift_right_logical(lax.iota(jnp.int32, 16), 2)
    def _sel_f(v0, v1, v2, v3):
        return jnp.where(lvl == 0, v0,
                         jnp.where(lvl == 1, v1,
                                   jnp.where(lvl == 2, v2, v3)))
    wf = _sel_f(64.0, 32.0, 16.0, 8.0)
    hf = _sel_f(64.0, 32.0, 16.0, 8.0)
    wi8 = _sel_f(64 * 8, 32 * 8, 16 * 8, 8 * 8).astype(jnp.int32)
    ls8 = _sel_f(0.0, 4096.0 * 8, 5120.0 * 8, 5376.0 * 8).astype(jnp.int32)
    zero = jnp.zeros((16,), jnp.float32)
    one = jnp.full((16,), 1.0, jnp.float32)

    def in_start(g, p):
        q = q0 + g * CQ
        soff = (b * LQ + q) * D_MODEL
        aoff = (b * LQ + q) * (N_HEADS * NPT)
        pltpu.async_copy(sloc_hbm.at[pl.ds(soff, CQ * D_MODEL)],
                         sloc_v.at[p], sem_in[p])
        pltpu.async_copy(aw_hbm.at[pl.ds(aoff, CQ * N_HEADS * NPT)],
                         aw_v.at[p], sem_in[p])

    def in_wait(p):
        pltpu.make_async_copy(sloc_hbm.at[pl.ds(0, CQ * D_MODEL)],
                              sloc_v.at[p], sem_in[p]).wait()
        pltpu.make_async_copy(aw_hbm.at[pl.ds(0, CQ * N_HEADS * NPT)],
                              aw_v.at[p], sem_in[p]).wait()

    def fire(g, p):
        def qh_step(qh, carry):
                h = qh % N_HEADS
                base = qh * 64
                x = sloc_v[p, pl.ds(qh * 2 * NPT, 16)]
                y = sloc_v[p, pl.ds(qh * 2 * NPT + NPT, 16)]
                awv = aw_v[p, pl.ds(qh * NPT, 16)]
                px = x * wf - 0.5
                py = y * hf - 0.5
                tx = px.astype(jnp.int32).astype(jnp.float32)
                x0 = jnp.where(tx > px, tx - 1.0, tx)
                ty = py.astype(jnp.int32).astype(jnp.float32)
                y0 = jnp.where(ty > py, ty - 1.0, ty)
                fx = px - x0
                fy = py - y0
                x1 = x0 + 1.0
                y1 = y0 + 1.0
                vx0 = (x0 >= 0.0) & (x0 < wf)
                vx1 = (x1 >= 0.0) & (x1 < wf)
                vy0 = (y0 >= 0.0) & (y0 < hf)
                vy1 = (y1 >= 0.0) & (y1 < hf)
                xi0 = jnp.clip(x0, 0.0, wf - 1.0).astype(jnp.int32) * N_HEADS
                xi1 = jnp.clip(x1, 0.0, wf - 1.0).astype(jnp.int32) * N_HEADS
                yi0 = jnp.clip(y0, 0.0, hf - 1.0).astype(jnp.int32)
                yi1 = jnp.clip(y1, 0.0, hf - 1.0).astype(jnp.int32)
                rbase = (b * LQ * N_HEADS + h) + ls8
                r0 = rbase + yi0 * wi8
                r1 = rbase + yi1 * wi8
                idx_v[p, pl.ds(base, 16)] = r0 + xi0
                idx_v[p, pl.ds(base + 16, 16)] = r0 + xi1
                idx_v[p, pl.ds(base + 32, 16)] = r1 + xi0
                idx_v[p, pl.ds(base + 48, 16)] = r1 + xi1
                wx0 = one - fx
                wy0 = one - fy
                a0 = awv * wy0
                a1 = awv * fy
                w_v[p, pl.ds(base, 16)] = jnp.where(vx0 & vy0, a0 * wx0, zero)
                w_v[p, pl.ds(base + 16, 16)] = jnp.where(vx1 & vy0, a0 * fx, zero)
                w_v[p, pl.ds(base + 32, 16)] = jnp.where(vx0 & vy1, a1 * wx0, zero)
                w_v[p, pl.ds(base + 48, 16)] = jnp.where(vx1 & vy1, a1 * fx, zero)
                return carry

        lax.fori_loop(0, CHQH, qh_step, 0)

        pltpu.async_copy(
            val_hbm.at[idx_v.at[p]], rows_v.at[p], sem_g[p])

    def gwait(p):
        pltpu.make_async_copy(
            val_hbm.at[idx_v.at[p]], rows_v.at[p], sem_g[p]).wait()

    def accum(g, p):
        gwait(p)

        def acc_qh(qh, carry):
            base = qh * 64
            a0 = zero
            a1 = zero
            for c in range(4):
                wvec = w_v[p, pl.ds(base + c * 16, 16)]
                for j in range(16):
                    r = base + c * 16 + j
                    w = wvec[j]
                    a0 = a0 + w * rows_v[p, r, pl.ds(0, 16)]
                    a1 = a1 + w * rows_v[p, r, pl.ds(16, 16)]
            out_v[p, pl.ds(qh * DH, 16)] = a0
            out_v[p, pl.ds(qh * DH + 16, 16)] = a1
            return carry

        lax.fori_loop(0, CHQH, acc_qh, 0)
        q = q0 + g * CQ
        soff = (b * LQ + q) * D_MODEL
        pltpu.async_copy(out_v.at[p], out_hbm.at[pl.ds(soff, CQ * D_MODEL)],
                         sem_o[p])

    def out_wait(p):
        pltpu.make_async_copy(out_v.at[p], out_hbm.at[pl.ds(0, CQ * D_MODEL)],
                              sem_o[p]).wait()

    in_start(0, 0)
    in_wait(0)
    fire(0, 0)
    in_start(1, 1)

    def body(i, carry):
        g0 = 2 * i
        g1 = g0 + 1
        in_wait(1)
        fire(g1, 1)
        in_start(jnp.minimum(g0 + 2, NCH - 2), 0)

        @pl.when(i > 0)
        def _():
            out_wait(0)

        accum(g0, 0)
        in_wait(0)
        fire(jnp.minimum(g0 + 2, NCH - 2), 0)
        in_start(jnp.minimum(g1 + 2, NCH - 1), 1)

        @pl.when(i > 0)
        def _():
            out_wait(1)

        accum(g1, 1)
        return carry

    lax.fori_loop(0, NCH // 2, body, 0)
    gwait(0)
    in_wait(1)
    out_wait(0)
    out_wait(1)


@functools.cache
def _make_sample_call():
    return pl.kernel(
        _sample_body,
        out_type=jax.ShapeDtypeStruct((B * LQ * D_MODEL,), jnp.float32),
        mesh=plsc.VectorSubcoreMesh(
            core_axis_name="c", subcore_axis_name="s",
            num_cores=2, num_subcores=16),
        compiler_params=pltpu.CompilerParams(use_tc_tiling_on_sc=False),
        scratch_types=[
            pltpu.VMEM((2, ROWS), jnp.int32),
            pltpu.VMEM((2, ROWS), jnp.float32),
            pltpu.VMEM((2, ROWS, DH), jnp.float32),
            pltpu.VMEM((2, CQ * D_MODEL), jnp.float32),
            pltpu.VMEM((2, CQ * N_HEADS * NPT), jnp.float32),
            pltpu.VMEM((2, CQ * D_MODEL), jnp.float32),
            pltpu.SemaphoreType.DMA,
            pltpu.SemaphoreType.DMA,
            pltpu.SemaphoreType.DMA,
            pltpu.SemaphoreType.DMA,
            pltpu.SemaphoreType.DMA,
            pltpu.SemaphoreType.DMA,
        ],
    )


def _sample_call(val, sloc, aw):
    return _make_sample_call()(val, sloc, aw)


def _ref_points_flat(valid_ratios):
    ref_list = []
    for lvl, (H_, W_) in enumerate(SHAPES):
        ry, rx = jnp.meshgrid(jnp.linspace(0.5, H_ - 0.5, H_),
                              jnp.linspace(0.5, W_ - 0.5, W_), indexing='ij')
        ry = ry.reshape(-1)[None] / (valid_ratios[:, None, lvl, 1] * H_)
        rx = rx.reshape(-1)[None] / (valid_ratios[:, None, lvl, 0] * W_)
        ref_list.append(jnp.stack((rx, ry), -1))
    rp = jnp.concatenate(ref_list, 1)
    rp = rp[:, :, None] * valid_ratios[:, None]
    rp = rp.transpose(0, 1, 3, 2)
    rp = jnp.repeat(rp, N_POINTS, axis=-1)
    rp = jnp.broadcast_to(rp[:, :, None], (B, LQ, N_HEADS, 2, NPT))
    return rp.reshape(B, LQ, D_MODEL)


def kernel(src, spatial_shapes, level_start_index, valid_ratios, pos, params):
    rp_flat = _ref_points_flat(valid_ratios)

    inv = np.zeros((N_HEADS, 2, NPT), np.float32)
    inv[:, 0, :] = 1.0 / _W
    inv[:, 1, :] = 1.0 / _H
    inv_flat = jnp.asarray(inv.reshape(D_MODEL))
    wo_p = params['W_off'].reshape(N_LAYERS, D_MODEL, N_HEADS, N_LEVELS, N_POINTS, 2)
    wo_p = wo_p.transpose(0, 1, 2, 5, 3, 4).reshape(N_LAYERS, D_MODEL, D_MODEL) * inv_flat
    bo_p = params['b_off'].reshape(N_LAYERS, N_HEADS, N_LEVELS, N_POINTS, 2)
    bo_p = bo_p.transpose(0, 1, 4, 2, 3).reshape(N_LAYERS, 1, D_MODEL) * inv_flat

    out = src
    sl_all, aw_all = [], []
    for lid in range(N_LAYERS):
        p = {k: v[lid] for k, v in params.items()}
        val, sloc_p, aw = _proj_call(
            out, pos, rp_flat,
            p['W_value'], p['b_value'][None],
            wo_p[lid], bo_p[lid],
            p['W_attn'], p['b_attn'][None])
        attn_flat = _sample_call(
            val.reshape(B * LQ * N_HEADS, DH),
            sloc_p.reshape(-1), aw.reshape(-1))
        out = _post_call(
            attn_flat.reshape(B, LQ, D_MODEL), out,
            p['W_out'], p['b_out'][None],
            p['ln1_g'][None], p['ln1_b'][None],
            p['W_ff1'], p['b_ff1'][None],
            p['W_ff2'], p['b_ff2'][None],
            p['ln2_g'][None], p['ln2_b'][None])
        sl_all.append(sloc_p)
        aw_all.append(aw)

    sl = jnp.stack(sl_all, 1).reshape(B, N_LAYERS, LQ, N_HEADS, 2, N_LEVELS, N_POINTS)
    sl = sl.transpose(0, 1, 2, 3, 5, 6, 4)
    aw = jnp.stack(aw_all, 1).reshape(B, N_LAYERS, LQ, N_HEADS, N_LEVELS, N_POINTS)
    return out, sl, aw


# speedup vs baseline: 1.2084x; 1.2084x over previous
import functools

import jax
import jax.numpy as jnp
import numpy as np
from jax import lax
from jax.experimental import pallas as pl
from jax.experimental.pallas import tpu as pltpu
from jax.experimental.pallas import tpu_sc as plsc

D_MODEL = 256
N_HEADS = 8
N_LEVELS = 4
N_POINTS = 4
N_LAYERS = 6
D_FFN = 1024
SHAPES = [(64, 64), (32, 32), (16, 16), (8, 8)]
LQ = sum(h * w for h, w in SHAPES)
B = 2
DH = D_MODEL // N_HEADS
NPT = N_LEVELS * N_POINTS

NB = 10
T = LQ // NB

N_TILES = 32
TILES_PER_B = N_TILES // B
QPT = LQ // TILES_PER_B
CQ = 5
NCH = QPT // CQ
CHQH = CQ * N_HEADS
ROWS = CHQH * 64

_DH_ORDER = np.concatenate([np.arange(0, DH, 2), np.arange(1, DH, 2)])
_ATTN_PERM = (np.arange(N_HEADS)[:, None] * DH + _DH_ORDER[None, :]).reshape(-1)

_LVL = np.repeat(np.arange(N_LEVELS), N_POINTS)
_W = np.array([s[1] for s in SHAPES], np.float32)[_LVL]
_H = np.array([s[0] for s in SHAPES], np.float32)[_LVL]
_LS = np.array([0, 4096, 5120, 5376], np.int32)[_LVL]


def _lane_const_f(vals):
    return jnp.asarray(vals, jnp.float32)


def _lane_const_i(vals):
    return jnp.asarray(vals, jnp.int32)


def _proj_body(src_ref, pos_ref, rp_ref, wv_ref, bv_ref, wo_ref, bo_ref,
               wa_ref, ba_ref, val_ref, sloc_ref, aw_ref):
    s = src_ref[0]
    q = s + pos_ref[0]
    val_ref[0] = (jnp.dot(s, wv_ref[...], preferred_element_type=jnp.float32)
                  + bv_ref[0]).astype(jnp.bfloat16)
    sloc_ref[0] = rp_ref[0] + jnp.dot(
        q, wo_ref[...], preferred_element_type=jnp.float32) + bo_ref[0]
    logits = jnp.dot(q, wa_ref[...], preferred_element_type=jnp.float32) + ba_ref[0]
    lg = logits.reshape(T, N_HEADS, NPT)
    m = jnp.max(lg, axis=-1, keepdims=True)
    e = jnp.exp(lg - m)
    aw = e / jnp.sum(e, axis=-1, keepdims=True)
    aw_ref[0] = aw.reshape(T, N_HEADS * NPT)


def _proj_call(src, pos, rp_flat, wv, bv, wo_p, bo_p, wa, ba):
    tok = lambda b, i: (b, i, 0)
    fixed = lambda b, i: (0, 0)
    return pl.pallas_call(
        _proj_body,
        grid=(B, NB),
        in_specs=[
            pl.BlockSpec((1, T, D_MODEL), tok),
            pl.BlockSpec((1, T, D_MODEL), tok),
            pl.BlockSpec((1, T, D_MODEL), tok),
            pl.BlockSpec((D_MODEL, D_MODEL), fixed),
            pl.BlockSpec((1, D_MODEL), fixed),
            pl.BlockSpec((D_MODEL, D_MODEL), fixed),
            pl.BlockSpec((1, D_MODEL), fixed),
            pl.BlockSpec((D_MODEL, N_HEADS * NPT), fixed),
            pl.BlockSpec((1, N_HEADS * NPT), fixed),
        ],
        out_specs=[
            pl.BlockSpec((1, T, D_MODEL), tok),
            pl.BlockSpec((1, T, D_MODEL), tok),
            pl.BlockSpec((1, T, N_HEADS * NPT), tok),
        ],
        out_shape=[
            jax.ShapeDtypeStruct((B, LQ, D_MODEL), jnp.bfloat16),
            jax.ShapeDtypeStruct((B, LQ, D_MODEL), jnp.float32),
            jax.ShapeDtypeStruct((B, LQ, N_HEADS * NPT), jnp.float32),
        ],
    )(src, pos, rp_flat, wv, bv, wo_p, bo_p, wa, ba)


def _post_body(attn_ref, src_ref, wo_ref, bo_ref, g1_ref, b1_ref,
               w1_ref, bf1_ref, w2_ref, bf2_ref, g2_ref, b2_ref, out_ref):
    a = (jnp.dot(attn_ref[0], wo_ref[...], preferred_element_type=jnp.float32)
         + bo_ref[0] + src_ref[0])
    mu = jnp.mean(a, axis=-1, keepdims=True)
    var = jnp.mean(jnp.square(a - mu), axis=-1, keepdims=True)
    s2 = (a - mu) * jax.lax.rsqrt(var + 1e-5) * g1_ref[0] + b1_ref[0]
    h = jnp.maximum(
        jnp.dot(s2, w1_ref[...], preferred_element_type=jnp.float32) + bf1_ref[0], 0.0)
    f = (jnp.dot(h, w2_ref[...], preferred_element_type=jnp.float32)
         + bf2_ref[0] + s2)
    mu2 = jnp.mean(f, axis=-1, keepdims=True)
    var2 = jnp.mean(jnp.square(f - mu2), axis=-1, keepdims=True)
    out_ref[0] = (f - mu2) * jax.lax.rsqrt(var2 + 1e-5) * g2_ref[0] + b2_ref[0]


def _post_call(attn, src, wo, bo, g1, b1, w1, bf1, w2, bf2, g2, b2):
    tok = lambda b, i: (b, i, 0)
    fixed = lambda b, i: (0, 0)
    return pl.pallas_call(
        _post_body,
        grid=(B, NB),
        in_specs=[
            pl.BlockSpec((1, T, D_MODEL), tok),
            pl.BlockSpec((1, T, D_MODEL), tok),
            pl.BlockSpec((D_MODEL, D_MODEL), fixed),
            pl.BlockSpec((1, D_MODEL), fixed),
            pl.BlockSpec((1, D_MODEL), fixed),
            pl.BlockSpec((1, D_MODEL), fixed),
            pl.BlockSpec((D_MODEL, D_FFN), fixed),
            pl.BlockSpec((1, D_FFN), fixed),
            pl.BlockSpec((D_FFN, D_MODEL), fixed),
            pl.BlockSpec((1, D_MODEL), fixed),
            pl.BlockSpec((1, D_MODEL), fixed),
            pl.BlockSpec((1, D_MODEL), fixed),
        ],
        out_specs=pl.BlockSpec((1, T, D_MODEL), tok),
        out_shape=jax.ShapeDtypeStruct((B, LQ, D_MODEL), jnp.float32),
    )(attn, src, wo, bo, g1, b1, w1, bf1, w2, bf2, g2, b2)


def _sample_body(val_hbm, sloc_hbm, aw_hbm, out_hbm,
                 idx_v, w_v, rows_v, sloc_v, aw_v, out_v,
                 sem_in0, sem_in1, sem_g0, sem_g1, sem_o0, sem_o1):
    sem_in = (sem_in0, sem_in1)
    sem_g = (sem_g0, sem_g1)
    sem_o = (sem_o0, sem_o1)
    wid = lax.axis_index("s") * 2 + lax.axis_index("c")
    b = wid % 2
    q0 = (wid // 2) * QPT

    lvl = lax.shift_right_logical(lax.iota(jnp.int32, 16), 2)
    def _sel_f(v0, v1, v2, v3):
        return jnp.where(lvl == 0, v0,
                         jnp.where(lvl == 1, v1,
                                   jnp.where(lvl == 2, v2, v3)))
    wf = _sel_f(64.0, 32.0, 16.0, 8.0)
    hf = _sel_f(64.0, 32.0, 16.0, 8.0)
    wi8 = _sel_f(64 * 8, 32 * 8, 16 * 8, 8 * 8).astype(jnp.int32)
    ls8 = _sel_f(0.0, 4096.0 * 8, 5120.0 * 8, 5376.0 * 8).astype(jnp.int32)
    zero = jnp.zeros((16,), jnp.float32)
    one = jnp.full((16,), 1.0, jnp.float32)

    def in_start(g, p):
        q = q0 + g * CQ
        soff = (b * LQ + q) * D_MODEL
        aoff = (b * LQ + q) * (N_HEADS * NPT)
        pltpu.async_copy(sloc_hbm.at[pl.ds(soff, CQ * D_MODEL)],
                         sloc_v.at[p], sem_in[p])
        pltpu.async_copy(aw_hbm.at[pl.ds(aoff, CQ * N_HEADS * NPT)],
                         aw_v.at[p], sem_in[p])

    def in_wait(p):
        pltpu.make_async_copy(sloc_hbm.at[pl.ds(0, CQ * D_MODEL)],
                              sloc_v.at[p], sem_in[p]).wait()
        pltpu.make_async_copy(aw_hbm.at[pl.ds(0, CQ * N_HEADS * NPT)],
                              aw_v.at[p], sem_in[p]).wait()

    def fire(g, p):
        def qh_step(qh, carry):
                h = qh % N_HEADS
                base = qh * 64
                x = sloc_v[p, pl.ds(qh * 2 * NPT, 16)]
                y = sloc_v[p, pl.ds(qh * 2 * NPT + NPT, 16)]
                awv = aw_v[p, pl.ds(qh * NPT, 16)]
                px = x * wf - 0.5
                py = y * hf - 0.5
                tx = px.astype(jnp.int32).astype(jnp.float32)
                x0 = jnp.where(tx > px, tx - 1.0, tx)
                ty = py.astype(jnp.int32).astype(jnp.float32)
                y0 = jnp.where(ty > py, ty - 1.0, ty)
                fx = px - x0
                fy = py - y0
                x1 = x0 + 1.0
                y1 = y0 + 1.0
                vx0 = (x0 >= 0.0) & (x0 < wf)
                vx1 = (x1 >= 0.0) & (x1 < wf)
                vy0 = (y0 >= 0.0) & (y0 < hf)
                vy1 = (y1 >= 0.0) & (y1 < hf)
                xi0 = jnp.clip(x0, 0.0, wf - 1.0).astype(jnp.int32) * N_HEADS
                xi1 = jnp.clip(x1, 0.0, wf - 1.0).astype(jnp.int32) * N_HEADS
                yi0 = jnp.clip(y0, 0.0, hf - 1.0).astype(jnp.int32)
                yi1 = jnp.clip(y1, 0.0, hf - 1.0).astype(jnp.int32)
                rbase = (b * LQ * N_HEADS + h) + ls8
                r0 = rbase + yi0 * wi8
                r1 = rbase + yi1 * wi8
                idx_v[p, pl.ds(base, 16)] = r0 + xi0
                idx_v[p, pl.ds(base + 16, 16)] = r0 + xi1
                idx_v[p, pl.ds(base + 32, 16)] = r1 + xi0
                idx_v[p, pl.ds(base + 48, 16)] = r1 + xi1
                wx0 = one - fx
                wy0 = one - fy
                a0 = awv * wy0
                a1 = awv * fy
                w_v[p, pl.ds(base, 16)] = jnp.where(vx0 & vy0, a0 * wx0, zero)
                w_v[p, pl.ds(base + 16, 16)] = jnp.where(vx1 & vy0, a0 * fx, zero)
                w_v[p, pl.ds(base + 32, 16)] = jnp.where(vx0 & vy1, a1 * wx0, zero)
                w_v[p, pl.ds(base + 48, 16)] = jnp.where(vx1 & vy1, a1 * fx, zero)
                return carry

        lax.fori_loop(0, CHQH, qh_step, 0)

        pltpu.async_copy(
            val_hbm.at[idx_v.at[p]], rows_v.at[p], sem_g[p])

    def gwait(p):
        pltpu.make_async_copy(
            val_hbm.at[idx_v.at[p]], rows_v.at[p], sem_g[p]).wait()

    def accum(g, p):
        gwait(p)

        def acc_qh(qh, carry):
            base = qh * 64
            a0 = zero
            a1 = zero
            for c in range(4):
                wvec = w_v[p, pl.ds(base + c * 16, 16)]
                for j in range(16):
                    r = base + c * 16 + j
                    w = wvec[j]
                    lo = rows_v[p, r, pl.ds(0, 16)].astype(jnp.float32)
                    hi = rows_v[p, r, pl.ds(16, 16)].astype(jnp.float32)
                    a0 = a0 + w * lo
                    a1 = a1 + w * hi
            out_v[p, pl.ds(qh * DH, 16)] = a0
            out_v[p, pl.ds(qh * DH + 16, 16)] = a1
            return carry

        lax.fori_loop(0, CHQH, acc_qh, 0)
        q = q0 + g * CQ
        soff = (b * LQ + q) * D_MODEL
        pltpu.async_copy(out_v.at[p], out_hbm.at[pl.ds(soff, CQ * D_MODEL)],
                         sem_o[p])

    def out_wait(p):
        pltpu.make_async_copy(out_v.at[p], out_hbm.at[pl.ds(0, CQ * D_MODEL)],
                              sem_o[p]).wait()

    in_start(0, 0)
    in_wait(0)
    fire(0, 0)
    in_start(1, 1)

    def body(i, carry):
        g0 = 2 * i
        g1 = g0 + 1
        in_wait(1)
        fire(g1, 1)
        in_start(jnp.minimum(g0 + 2, NCH - 2), 0)

        @pl.when(i > 0)
        def _():
            out_wait(0)

        accum(g0, 0)
        in_wait(0)
        fire(jnp.minimum(g0 + 2, NCH - 2), 0)
        in_start(jnp.minimum(g1 + 2, NCH - 1), 1)

        @pl.when(i > 0)
        def _():
            out_wait(1)

        accum(g1, 1)
        return carry

    lax.fori_loop(0, NCH // 2, body, 0)
    gwait(0)
    in_wait(1)
    out_wait(0)
    out_wait(1)


@functools.cache
def _make_sample_call():
    return pl.kernel(
        _sample_body,
        out_type=jax.ShapeDtypeStruct((B * LQ * D_MODEL,), jnp.float32),
        mesh=plsc.VectorSubcoreMesh(
            core_axis_name="c", subcore_axis_name="s",
            num_cores=2, num_subcores=16),
        compiler_params=pltpu.CompilerParams(use_tc_tiling_on_sc=False),
        scratch_types=[
            pltpu.VMEM((2, ROWS), jnp.int32),
            pltpu.VMEM((2, ROWS), jnp.float32),
            pltpu.VMEM((2, ROWS, DH), jnp.bfloat16),
            pltpu.VMEM((2, CQ * D_MODEL), jnp.float32),
            pltpu.VMEM((2, CQ * N_HEADS * NPT), jnp.float32),
            pltpu.VMEM((2, CQ * D_MODEL), jnp.float32),
            pltpu.SemaphoreType.DMA,
            pltpu.SemaphoreType.DMA,
            pltpu.SemaphoreType.DMA,
            pltpu.SemaphoreType.DMA,
            pltpu.SemaphoreType.DMA,
            pltpu.SemaphoreType.DMA,
        ],
    )


def _sample_call(val, sloc, aw):
    return _make_sample_call()(val, sloc, aw)


def _ref_points_flat(valid_ratios):
    ref_list = []
    for lvl, (H_, W_) in enumerate(SHAPES):
        ry, rx = jnp.meshgrid(jnp.linspace(0.5, H_ - 0.5, H_),
                              jnp.linspace(0.5, W_ - 0.5, W_), indexing='ij')
        ry = ry.reshape(-1)[None] / (valid_ratios[:, None, lvl, 1] * H_)
        rx = rx.reshape(-1)[None] / (valid_ratios[:, None, lvl, 0] * W_)
        ref_list.append(jnp.stack((rx, ry), -1))
    rp = jnp.concatenate(ref_list, 1)
    rp = rp[:, :, None] * valid_ratios[:, None]
    rp = rp.transpose(0, 1, 3, 2)
    rp = jnp.repeat(rp, N_POINTS, axis=-1)
    rp = jnp.broadcast_to(rp[:, :, None], (B, LQ, N_HEADS, 2, NPT))
    return rp.reshape(B, LQ, D_MODEL)


def kernel(src, spatial_shapes, level_start_index, valid_ratios, pos, params):
    rp_flat = _ref_points_flat(valid_ratios)

    inv = np.zeros((N_HEADS, 2, NPT), np.float32)
    inv[:, 0, :] = 1.0 / _W
    inv[:, 1, :] = 1.0 / _H
    inv_flat = jnp.asarray(inv.reshape(D_MODEL))
    wo_p = params['W_off'].reshape(N_LAYERS, D_MODEL, N_HEADS, N_LEVELS, N_POINTS, 2)
    wo_p = wo_p.transpose(0, 1, 2, 5, 3, 4).reshape(N_LAYERS, D_MODEL, D_MODEL) * inv_flat
    bo_p = params['b_off'].reshape(N_LAYERS, N_HEADS, N_LEVELS, N_POINTS, 2)
    bo_p = bo_p.transpose(0, 1, 4, 2, 3).reshape(N_LAYERS, 1, D_MODEL) * inv_flat
    wout_p = params['W_out']

    out = src
    sl_all, aw_all = [], []
    for lid in range(N_LAYERS):
        p = {k: v[lid] for k, v in params.items()}
        val, sloc_p, aw = _proj_call(
            out, pos, rp_flat,
            p['W_value'], p['b_value'][None],
            wo_p[lid], bo_p[lid],
            p['W_attn'], p['b_attn'][None])
        attn_flat = _sample_call(
            val.reshape(B * LQ * N_HEADS, DH),
            sloc_p.reshape(-1), aw.reshape(-1))
        out = _post_call(
            attn_flat.reshape(B, LQ, D_MODEL), out,
            wout_p[lid], p['b_out'][None],
            p['ln1_g'][None], p['ln1_b'][None],
            p['W_ff1'], p['b_ff1'][None],
            p['W_ff2'], p['b_ff2'][None],
            p['ln2_g'][None], p['ln2_b'][None])
        sl_all.append(sloc_p)
        aw_all.append(aw)

    sl = jnp.stack(sl_all, 1).reshape(B, N_LAYERS, LQ, N_HEADS, 2, N_LEVELS, N_POINTS)
    sl = sl.transpose(0, 1, 2, 3, 5, 6, 4)
    aw = jnp.stack(aw_all, 1).reshape(B, N_LAYERS, LQ, N_HEADS, N_LEVELS, N_POINTS)
    return out, sl, aw


# speedup vs baseline: 1.2088x; 1.0004x over previous
import functools

import jax
import jax.numpy as jnp
import numpy as np
from jax import lax
from jax.experimental import pallas as pl
from jax.experimental.pallas import tpu as pltpu
from jax.experimental.pallas import tpu_sc as plsc

D_MODEL = 256
N_HEADS = 8
N_LEVELS = 4
N_POINTS = 4
N_LAYERS = 6
D_FFN = 1024
SHAPES = [(64, 64), (32, 32), (16, 16), (8, 8)]
LQ = sum(h * w for h, w in SHAPES)
B = 2
DH = D_MODEL // N_HEADS
NPT = N_LEVELS * N_POINTS

NB = 10
T = LQ // NB

N_TILES = 32
TILES_PER_B = N_TILES // B
QPT = LQ // TILES_PER_B
CQ = 5
NCH = QPT // CQ
CHQH = CQ * N_HEADS
ROWS = CHQH * 64

_DH_ORDER = np.concatenate([np.arange(0, DH, 2), np.arange(1, DH, 2)])
_ATTN_PERM = (np.arange(N_HEADS)[:, None] * DH + _DH_ORDER[None, :]).reshape(-1)

_LVL = np.repeat(np.arange(N_LEVELS), N_POINTS)
_W = np.array([s[1] for s in SHAPES], np.float32)[_LVL]
_H = np.array([s[0] for s in SHAPES], np.float32)[_LVL]
_LS = np.array([0, 4096, 5120, 5376], np.int32)[_LVL]


def _lane_const_f(vals):
    return jnp.asarray(vals, jnp.float32)


def _lane_const_i(vals):
    return jnp.asarray(vals, jnp.int32)


def _proj_body(src_ref, pos_ref, rp_ref, wv_ref, bv_ref, wo_ref, bo_ref,
               wa_ref, ba_ref, val_ref, sloc_ref, aw_ref):
    s = src_ref[0]
    q = s + pos_ref[0]
    val_ref[0] = (jnp.dot(s.astype(jnp.bfloat16), wv_ref[...].astype(jnp.bfloat16),
                          preferred_element_type=jnp.float32)
                  + bv_ref[0]).astype(jnp.bfloat16)
    sloc_ref[0] = rp_ref[0] + jnp.dot(
        q, wo_ref[...], preferred_element_type=jnp.float32) + bo_ref[0]
    logits = jnp.dot(q, wa_ref[...], preferred_element_type=jnp.float32) + ba_ref[0]
    lg = logits.reshape(T, N_HEADS, NPT)
    m = jnp.max(lg, axis=-1, keepdims=True)
    e = jnp.exp(lg - m)
    aw = e / jnp.sum(e, axis=-1, keepdims=True)
    aw_ref[0] = aw.reshape(T, N_HEADS * NPT)


def _proj_call(src, pos, rp_flat, wv, bv, wo_p, bo_p, wa, ba):
    tok = lambda b, i: (b, i, 0)
    fixed = lambda b, i: (0, 0)
    return pl.pallas_call(
        _proj_body,
        grid=(B, NB),
        in_specs=[
            pl.BlockSpec((1, T, D_MODEL), tok),
            pl.BlockSpec((1, T, D_MODEL), tok),
            pl.BlockSpec((1, T, D_MODEL), tok),
            pl.BlockSpec((D_MODEL, D_MODEL), fixed),
            pl.BlockSpec((1, D_MODEL), fixed),
            pl.BlockSpec((D_MODEL, D_MODEL), fixed),
            pl.BlockSpec((1, D_MODEL), fixed),
            pl.BlockSpec((D_MODEL, N_HEADS * NPT), fixed),
            pl.BlockSpec((1, N_HEADS * NPT), fixed),
        ],
        out_specs=[
            pl.BlockSpec((1, T, D_MODEL), tok),
            pl.BlockSpec((1, T, D_MODEL), tok),
            pl.BlockSpec((1, T, N_HEADS * NPT), tok),
        ],
        out_shape=[
            jax.ShapeDtypeStruct((B, LQ, D_MODEL), jnp.bfloat16),
            jax.ShapeDtypeStruct((B, LQ, D_MODEL), jnp.float32),
            jax.ShapeDtypeStruct((B, LQ, N_HEADS * NPT), jnp.float32),
        ],
    )(src, pos, rp_flat, wv, bv, wo_p, bo_p, wa, ba)


def _post_body(attn_ref, src_ref, wo_ref, bo_ref, g1_ref, b1_ref,
               w1_ref, bf1_ref, w2_ref, bf2_ref, g2_ref, b2_ref, out_ref):
    a = (jnp.dot(attn_ref[0], wo_ref[...], preferred_element_type=jnp.float32)
         + bo_ref[0] + src_ref[0])
    mu = jnp.mean(a, axis=-1, keepdims=True)
    var = jnp.mean(jnp.square(a - mu), axis=-1, keepdims=True)
    s2 = (a - mu) * jax.lax.rsqrt(var + 1e-5) * g1_ref[0] + b1_ref[0]
    h = jnp.maximum(
        jnp.dot(s2.astype(jnp.bfloat16), w1_ref[...].astype(jnp.bfloat16),
                preferred_element_type=jnp.float32) + bf1_ref[0], 0.0)
    f = (jnp.dot(h.astype(jnp.bfloat16), w2_ref[...].astype(jnp.bfloat16),
                 preferred_element_type=jnp.float32)
         + bf2_ref[0] + s2)
    mu2 = jnp.mean(f, axis=-1, keepdims=True)
    var2 = jnp.mean(jnp.square(f - mu2), axis=-1, keepdims=True)
    out_ref[0] = (f - mu2) * jax.lax.rsqrt(var2 + 1e-5) * g2_ref[0] + b2_ref[0]


def _post_call(attn, src, wo, bo, g1, b1, w1, bf1, w2, bf2, g2, b2):
    tok = lambda b, i: (b, i, 0)
    fixed = lambda b, i: (0, 0)
    return pl.pallas_call(
        _post_body,
        grid=(B, NB),
        in_specs=[
            pl.BlockSpec((1, T, D_MODEL), tok),
            pl.BlockSpec((1, T, D_MODEL), tok),
            pl.BlockSpec((D_MODEL, D_MODEL), fixed),
            pl.BlockSpec((1, D_MODEL), fixed),
            pl.BlockSpec((1, D_MODEL), fixed),
            pl.BlockSpec((1, D_MODEL), fixed),
            pl.BlockSpec((D_MODEL, D_FFN), fixed),
            pl.BlockSpec((1, D_FFN), fixed),
            pl.BlockSpec((D_FFN, D_MODEL), fixed),
            pl.BlockSpec((1, D_MODEL), fixed),
            pl.BlockSpec((1, D_MODEL), fixed),
            pl.BlockSpec((1, D_MODEL), fixed),
        ],
        out_specs=pl.BlockSpec((1, T, D_MODEL), tok),
        out_shape=jax.ShapeDtypeStruct((B, LQ, D_MODEL), jnp.float32),
    )(attn, src, wo, bo, g1, b1, w1, bf1, w2, bf2, g2, b2)


def _sample_body(val_hbm, sloc_hbm, aw_hbm, out_hbm,
                 idx_v, w_v, rows_v, sloc_v, aw_v, out_v,
                 sem_in0, sem_in1, sem_g0, sem_g1, sem_o0, sem_o1):
    sem_in = (sem_in0, sem_in1)
    sem_g = (sem_g0, sem_g1)
    sem_o = (sem_o0, sem_o1)
    wid = lax.axis_index("s") * 2 + lax.axis_index("c")
    b = wid % 2
    q0 = (wid // 2) * QPT

    lvl = lax.shift_right_logical(lax.iota(jnp.int32, 16), 2)
    def _sel_f(v0, v1, v2, v3):
        return jnp.where(lvl == 0, v0,
                         jnp.where(lvl == 1, v1,
                                   jnp.where(lvl == 2, v2, v3)))
    wf = _sel_f(64.0, 32.0, 16.0, 8.0)
    hf = _sel_f(64.0, 32.0, 16.0, 8.0)
    wi8 = _sel_f(64 * 8, 32 * 8, 16 * 8, 8 * 8).astype(jnp.int32)
    ls8 = _sel_f(0.0, 4096.0 * 8, 5120.0 * 8, 5376.0 * 8).astype(jnp.int32)
    zero = jnp.zeros((16,), jnp.float32)
    one = jnp.full((16,), 1.0, jnp.float32)

    def in_start(g, p):
        q = q0 + g * CQ
        soff = (b * LQ + q) * D_MODEL
        aoff = (b * LQ + q) * (N_HEADS * NPT)
        pltpu.async_copy(sloc_hbm.at[pl.ds(soff, CQ * D_MODEL)],
                         sloc_v.at[p], sem_in[p])
        pltpu.async_copy(aw_hbm.at[pl.ds(aoff, CQ * N_HEADS * NPT)],
                         aw_v.at[p], sem_in[p])

    def in_wait(p):
        pltpu.make_async_copy(sloc_hbm.at[pl.ds(0, CQ * D_MODEL)],
                              sloc_v.at[p], sem_in[p]).wait()
        pltpu.make_async_copy(aw_hbm.at[pl.ds(0, CQ * N_HEADS * NPT)],
                              aw_v.at[p], sem_in[p]).wait()

    def fire(g, p):
        def qh_step(qh, carry):
                h = qh % N_HEADS
                base = qh * 64
                x = sloc_v[p, pl.ds(qh * 2 * NPT, 16)]
                y = sloc_v[p, pl.ds(qh * 2 * NPT + NPT, 16)]
                awv = aw_v[p, pl.ds(qh * NPT, 16)]
                px = x * wf - 0.5
                py = y * hf - 0.5
                tx = px.astype(jnp.int32).astype(jnp.float32)
                x0 = jnp.where(tx > px, tx - 1.0, tx)
                ty = py.astype(jnp.int32).astype(jnp.float32)
                y0 = jnp.where(ty > py, ty - 1.0, ty)
                fx = px - x0
                fy = py - y0
                x1 = x0 + 1.0
                y1 = y0 + 1.0
                vx0 = (x0 >= 0.0) & (x0 < wf)
                vx1 = (x1 >= 0.0) & (x1 < wf)
                vy0 = (y0 >= 0.0) & (y0 < hf)
                vy1 = (y1 >= 0.0) & (y1 < hf)
                xi0 = jnp.clip(x0, 0.0, wf - 1.0).astype(jnp.int32) * N_HEADS
                xi1 = jnp.clip(x1, 0.0, wf - 1.0).astype(jnp.int32) * N_HEADS
                yi0 = jnp.clip(y0, 0.0, hf - 1.0).astype(jnp.int32)
                yi1 = jnp.clip(y1, 0.0, hf - 1.0).astype(jnp.int32)
                rbase = (b * LQ * N_HEADS + h) + ls8
                r0 = rbase + yi0 * wi8
                r1 = rbase + yi1 * wi8
                idx_v[p, pl.ds(base, 16)] = r0 + xi0
                idx_v[p, pl.ds(base + 16, 16)] = r0 + xi1
                idx_v[p, pl.ds(base + 32, 16)] = r1 + xi0
                idx_v[p, pl.ds(base + 48, 16)] = r1 + xi1
                wx0 = one - fx
                wy0 = one - fy
                a0 = awv * wy0
                a1 = awv * fy
                w_v[p, pl.ds(base, 16)] = jnp.where(vx0 & vy0, a0 * wx0, zero)
                w_v[p, pl.ds(base + 16, 16)] = jnp.where(vx1 & vy0, a0 * fx, zero)
                w_v[p, pl.ds(base + 32, 16)] = jnp.where(vx0 & vy1, a1 * wx0, zero)
                w_v[p, pl.ds(base + 48, 16)] = jnp.where(vx1 & vy1, a1 * fx, zero)
                return carry

        lax.fori_loop(0, CHQH, qh_step, 0)

        pltpu.async_copy(
            val_hbm.at[idx_v.at[p]], rows_v.at[p], sem_g[p])

    def gwait(p):
        pltpu.make_async_copy(
            val_hbm.at[idx_v.at[p]], rows_v.at[p], sem_g[p]).wait()

    def accum(g, p):
        gwait(p)

        def acc_qh(qh, carry):
            base = qh * 64
            a0 = zero
            a1 = zero
            for c in range(4):
                wvec = w_v[p, pl.ds(base + c * 16, 16)]
                for j in range(16):
                    r = base + c * 16 + j
                    w = wvec[j]
                    lo = rows_v[p, r, pl.ds(0, 16)].astype(jnp.float32)
                    hi = rows_v[p, r, pl.ds(16, 16)].astype(jnp.float32)
                    a0 = a0 + w * lo
                    a1 = a1 + w * hi
            out_v[p, pl.ds(qh * DH, 16)] = a0
            out_v[p, pl.ds(qh * DH + 16, 16)] = a1
            return carry

        lax.fori_loop(0, CHQH, acc_qh, 0)
        q = q0 + g * CQ
        soff = (b * LQ + q) * D_MODEL
        pltpu.async_copy(out_v.at[p], out_hbm.at[pl.ds(soff, CQ * D_MODEL)],
                         sem_o[p])

    def out_wait(p):
        pltpu.make_async_copy(out_v.at[p], out_hbm.at[pl.ds(0, CQ * D_MODEL)],
                              sem_o[p]).wait()

    in_start(0, 0)
    in_wait(0)
    fire(0, 0)
    in_start(1, 1)

    def body(i, carry):
        g0 = 2 * i
        g1 = g0 + 1
        in_wait(1)
        fire(g1, 1)
        in_start(jnp.minimum(g0 + 2, NCH - 2), 0)

        @pl.when(i > 0)
        def _():
            out_wait(0)

        accum(g0, 0)
        in_wait(0)
        fire(jnp.minimum(g0 + 2, NCH - 2), 0)
        in_start(jnp.minimum(g1 + 2, NCH - 1), 1)

        @pl.when(i > 0)
        def _():
            out_wait(1)

        accum(g1, 1)
        return carry

    lax.fori_loop(0, NCH // 2, body, 0)
    gwait(0)
    in_wait(1)
    out_wait(0)
    out_wait(1)


@functools.cache
def _make_sample_call():
    return pl.kernel(
        _sample_body,
        out_type=jax.ShapeDtypeStruct((B * LQ * D_MODEL,), jnp.float32),
        mesh=plsc.VectorSubcoreMesh(
            core_axis_name="c", subcore_axis_name="s",
            num_cores=2, num_subcores=16),
        compiler_params=pltpu.CompilerParams(use_tc_tiling_on_sc=False),
        scratch_types=[
            pltpu.VMEM((2, ROWS), jnp.int32),
            pltpu.VMEM((2, ROWS), jnp.float32),
            pltpu.VMEM((2, ROWS, DH), jnp.bfloat16),
            pltpu.VMEM((2, CQ * D_MODEL), jnp.float32),
            pltpu.VMEM((2, CQ * N_HEADS * NPT), jnp.float32),
            pltpu.VMEM((2, CQ * D_MODEL), jnp.float32),
            pltpu.SemaphoreType.DMA,
            pltpu.SemaphoreType.DMA,
            pltpu.SemaphoreType.DMA,
            pltpu.SemaphoreType.DMA,
            pltpu.SemaphoreType.DMA,
            pltpu.SemaphoreType.DMA,
        ],
    )


def _sample_call(val, sloc, aw):
    return _make_sample_call()(val, sloc, aw)


def _ref_points_flat(valid_ratios):
    ref_list = []
    for lvl, (H_, W_) in enumerate(SHAPES):
        ry, rx = jnp.meshgrid(jnp.linspace(0.5, H_ - 0.5, H_),
                              jnp.linspace(0.5, W_ - 0.5, W_), indexing='ij')
        ry = ry.reshape(-1)[None] / (valid_ratios[:, None, lvl, 1] * H_)
        rx = rx.reshape(-1)[None] / (valid_ratios[:, None, lvl, 0] * W_)
        ref_list.append(jnp.stack((rx, ry), -1))
    rp = jnp.concatenate(ref_list, 1)
    rp = rp[:, :, None] * valid_ratios[:, None]
    rp = rp.transpose(0, 1, 3, 2)
    rp = jnp.repeat(rp, N_POINTS, axis=-1)
    rp = jnp.broadcast_to(rp[:, :, None], (B, LQ, N_HEADS, 2, NPT))
    return rp.reshape(B, LQ, D_MODEL)


def kernel(src, spatial_shapes, level_start_index, valid_ratios, pos, params):
    rp_flat = _ref_points_flat(valid_ratios)

    inv = np.zeros((N_HEADS, 2, NPT), np.float32)
    inv[:, 0, :] = 1.0 / _W
    inv[:, 1, :] = 1.0 / _H
    inv_flat = jnp.asarray(inv.reshape(D_MODEL))
    wo_p = params['W_off'].reshape(N_LAYERS, D_MODEL, N_HEADS, N_LEVELS, N_POINTS, 2)
    wo_p = wo_p.transpose(0, 1, 2, 5, 3, 4).reshape(N_LAYERS, D_MODEL, D_MODEL) * inv_flat
    bo_p = params['b_off'].reshape(N_LAYERS, N_HEADS, N_LEVELS, N_POINTS, 2)
    bo_p = bo_p.transpose(0, 1, 4, 2, 3).reshape(N_LAYERS, 1, D_MODEL) * inv_flat
    wout_p = params['W_out']

    out = src
    sl_all, aw_all = [], []
    for lid in range(N_LAYERS):
        p = {k: v[lid] for k, v in params.items()}
        val, sloc_p, aw = _proj_call(
            out, pos, rp_flat,
            p['W_value'], p['b_value'][None],
            wo_p[lid], bo_p[lid],
            p['W_attn'], p['b_attn'][None])
        attn_flat = _sample_call(
            val.reshape(B * LQ * N_HEADS, DH),
            sloc_p.reshape(-1), aw.reshape(-1))
        out = _post_call(
            attn_flat.reshape(B, LQ, D_MODEL), out,
            wout_p[lid], p['b_out'][None],
            p['ln1_g'][None], p['ln1_b'][None],
            p['W_ff1'], p['b_ff1'][None],
            p['W_ff2'], p['b_ff2'][None],
            p['ln2_g'][None], p['ln2_b'][None])
        sl_all.append(sloc_p)
        aw_all.append(aw)

    sl = jnp.stack(sl_all, 1).reshape(B, N_LAYERS, LQ, N_HEADS, 2, N_LEVELS, N_POINTS)
    sl = sl.transpose(0, 1, 2, 3, 5, 6, 4)
    aw = jnp.stack(aw_all, 1).reshape(B, N_LAYERS, LQ, N_HEADS, N_LEVELS, N_POINTS)
    return out, sl, aw
